# baseline (speedup 1.0000x reference)
import functools

import jax
import jax.numpy as jnp
from jax import lax
from jax.experimental import pallas as pl
from jax.experimental.pallas import tpu as pltpu
from jax.experimental.pallas import tpu_sc as plsc

VOCAB = 100000
EMB = 128
B = 4096
CTX = 10

NC = 2
NS = 16
NW = NC * NS
BPW = B // NW
RC = 8
NCHUNK = BPW // RC
IDXW = RC * CTX

LANES = 16

BN = 512


def _pool_sc(x2d, table):
    mesh = plsc.VectorSubcoreMesh(core_axis_name="c", subcore_axis_name="s")

    @functools.partial(
        pl.kernel,
        mesh=mesh,
        out_type=jax.ShapeDtypeStruct((B, EMB), jnp.float32),
        scratch_types=[
            pltpu.VMEM((NCHUNK, IDXW), jnp.int32),
            pltpu.VMEM((IDXW, EMB), jnp.float32),
            pltpu.VMEM((RC, EMB), jnp.float32),
            pltpu.SemaphoreType.DMA,
        ],
    )
    def k(x_hbm, tab_hbm, out_hbm, idx_v, rows_v, pool_v, sem):
        wid = lax.axis_index("s") * NC + lax.axis_index("c")
        pltpu.sync_copy(x_hbm.at[pl.ds(wid * NCHUNK, NCHUNK)], idx_v)

        def chunk_body(kc, carry):
            pltpu.async_copy(tab_hbm.at[idx_v.at[kc]], rows_v, sem).wait()

            def row_body(r, carry2):
                base = r * CTX
                for g in range(EMB // LANES):
                    sl = pl.ds(g * LANES, LANES)
                    acc = rows_v[base, sl]
                    for j in range(1, CTX):
                        acc = acc + rows_v[base + j, sl]
                    pool_v[r, sl] = acc * (1.0 / CTX)
                return carry2

            lax.fori_loop(0, RC, row_body, 0)
            pltpu.sync_copy(pool_v, out_hbm.at[pl.ds(wid * BPW + kc * RC, RC)])
            return carry

        lax.fori_loop(0, NCHUNK, chunk_body, 0)

    return k(x2d, table)


def _matmul_tc(pooled, w, b2d):
    nt = pl.cdiv(VOCAB, BN)

    def mm(p_ref, w_ref, b_ref, o_ref):
        acc = jnp.dot(p_ref[...], w_ref[...], preferred_element_type=jnp.float32)
        o_ref[...] = acc + b_ref[...]

    return pl.pallas_call(
        mm,
        grid=(nt,),
        in_specs=[
            pl.BlockSpec((B, EMB), lambda j: (0, 0)),
            pl.BlockSpec((EMB, BN), lambda j: (0, j)),
            pl.BlockSpec((1, BN), lambda j: (0, j)),
        ],
        out_specs=pl.BlockSpec((B, BN), lambda j: (0, j)),
        out_shape=jax.ShapeDtypeStruct((B, VOCAB), jnp.float32),
    )(pooled, w, b2d)


def kernel(x, embed_table, W, b):
    x2d = x.reshape(NW * NCHUNK, IDXW).astype(jnp.int32)
    pooled = _pool_sc(x2d, embed_table)
    return _matmul_tc(
        pooled.astype(jnp.bfloat16),
        W.astype(jnp.bfloat16),
        b.reshape(1, VOCAB),
    )
